# Initial kernel scaffold; baseline (speedup 1.0000x reference)
import jax
import jax.numpy as jnp
from jax.experimental import pallas as pl


def kernel(x, table):
    raise NotImplementedError("write your pallas kernel here")


# speedup vs baseline: 13.5534x; 13.5534x over previous
import functools

import jax
import jax.numpy as jnp
from jax.experimental import pallas as pl
from jax.experimental.pallas import tpu as pltpu

_MAX_REL = 32
_NTAB = 2 * _MAX_REL + 1


def _rpe_kernel(table_ref, out_ref, g_ref, *, seq_len, d_model, rows_per_blk):
    @pl.when(pl.program_id(0) == 0)
    def _build_g():
        for k in range(8):
            lo = seq_len - 1 - _MAX_REL - k
            g_ref[k, 0:lo, :] = jnp.broadcast_to(table_ref[0:1, :], (lo, d_model))
            g_ref[k, lo:lo + _NTAB, :] = table_ref[:, :]
            tail = g_ref.shape[1] - (lo + _NTAB)
            g_ref[k, lo + _NTAB:, :] = jnp.broadcast_to(
                table_ref[_NTAB - 1:_NTAB, :], (tail, d_model))

    q = seq_len - rows_per_blk * (pl.program_id(0) + 1)
    q = pl.multiple_of(q, 8)
    for r in range(rows_per_blk):
        out_ref[r, :, :] = g_ref[7 - r, pl.ds(q, seq_len), :]


def kernel(x, table):
    seq_len = x.shape[1]
    d_model = table.shape[1]
    rows_per_blk = 8
    g_pad = 2 * seq_len

    body = functools.partial(
        _rpe_kernel, seq_len=seq_len, d_model=d_model, rows_per_blk=rows_per_blk)

    rel = pl.pallas_call(
        body,
        grid=(seq_len // rows_per_blk,),
        in_specs=[pl.BlockSpec((_NTAB, d_model), lambda i: (0, 0))],
        out_specs=pl.BlockSpec((rows_per_blk, seq_len, d_model),
                               lambda i: (i, 0, 0)),
        out_shape=jax.ShapeDtypeStruct((seq_len, seq_len, d_model), table.dtype),
        scratch_shapes=[pltpu.VMEM((8, g_pad, d_model), table.dtype)],
    )(table)
    return (x, rel)
